# Initial kernel scaffold; baseline (speedup 1.0000x reference)
import jax
import jax.numpy as jnp
from jax.experimental import pallas as pl


def kernel(x, edge_index, batch, proj_W, proj_b, conv0_W1, conv0_b1, conv0_W2, conv0_b2, conv1_W1, conv1_b1, conv1_W2, conv1_b2, conv2_W1, conv2_b1, conv2_W2, conv2_b2, shared_W, shared_b, energy_W, energy_b, dipole_W, dipole_b):
    raise NotImplementedError("write your pallas kernel here")


# speedup vs baseline: 3.7872x; 3.7872x over previous
import functools

import jax
import jax.numpy as jnp
from jax import lax
from jax.experimental import pallas as pl
from jax.experimental.pallas import tpu as pltpu
from jax.experimental.pallas import tpu_sc as plsc

_SC_PARAMS = pltpu.CompilerParams(use_tc_tiling_on_sc=False)

N = 50000
E = 800000
DIN = 4
D = 64
DH = 32
G = 512

NSUB = 16
NCORE = 2

NP = 51200
EPAD = 800768
CH = 128
ECHUNKS = EPAD // (NSUB * CH)
NCHUNKS = NP // (NSUB * CH)
GP = 528

BLK = 1024
NBLK = NP // BLK


def _sc_edge_agg(h_a, h_b, srcp, dstp):
    mesh = plsc.VectorSubcoreMesh(core_axis_name="c", subcore_axis_name="s")

    @functools.partial(
        pl.kernel,
        out_type=[jax.ShapeDtypeStruct((NP, DH), jnp.float32),
                  jax.ShapeDtypeStruct((NP, DH), jnp.float32)],
        mesh=mesh,
        compiler_params=_SC_PARAMS,
        scratch_types=[
            pltpu.VMEM_SHARED((NP, DH), jnp.float32),
            pltpu.VMEM((CH,), jnp.int32),
            pltpu.VMEM((1, CH), jnp.int32),
            pltpu.VMEM((CH, DH), jnp.float32),
            pltpu.VMEM((CH, DH), jnp.float32),
        ],
    )
    def k(ha_hbm, hb_hbm, src_hbm, dst_hbm, oa_hbm, ob_hbm,
          acc, idx_s, idx_d, rows, zbuf):
        c = lax.axis_index("c")
        s = lax.axis_index("s")

        @pl.loop(0, CH)
        def _(r):
            @pl.loop(0, DH // 16)
            def _(j):
                zbuf[r, pl.ds(j * 16, 16)] = jnp.zeros((16,), jnp.float32)

        @pl.loop(0, NCHUNKS)
        def _(kk):
            pltpu.sync_copy(zbuf, acc.at[pl.ds((s * NCHUNKS + kk) * CH, CH)])

        plsc.subcore_barrier()

        def edge_loop(h_hbm):
            base = s * (ECHUNKS * CH)

            @pl.loop(0, ECHUNKS)
            def _(kk):
                off = base + kk * CH
                pltpu.sync_copy(src_hbm.at[pl.ds(off, CH)], idx_s)
                pltpu.sync_copy(dst_hbm.at[pl.ds(off, CH)], idx_d.at[0])
                pltpu.sync_copy(h_hbm.at[idx_s], rows)
                pltpu.sync_copy(rows, acc.at[idx_d.at[0]], add=True)

        @pl.when(c == 0)
        def _():
            edge_loop(ha_hbm)

        @pl.when(c == 1)
        def _():
            edge_loop(hb_hbm)

        plsc.subcore_barrier()

        def copy_out(o_hbm):
            @pl.loop(0, NCHUNKS)
            def _(kk):
                r0 = (s * NCHUNKS + kk) * CH
                pltpu.sync_copy(acc.at[pl.ds(r0, CH)], rows)
                pltpu.sync_copy(rows, o_hbm.at[pl.ds(r0, CH)])

        @pl.when(c == 0)
        def _():
            copy_out(oa_hbm)

        @pl.when(c == 1)
        def _():
            copy_out(ob_hbm)

    return k(h_a, h_b, srcp, dstp)


def _sc_pool(h_a, h_b, batchp):
    mesh = plsc.VectorSubcoreMesh(core_axis_name="c", subcore_axis_name="s")

    @functools.partial(
        pl.kernel,
        out_type=[jax.ShapeDtypeStruct((G, DH), jnp.float32),
                  jax.ShapeDtypeStruct((G, DH), jnp.float32),
                  jax.ShapeDtypeStruct((G, DH), jnp.float32)],
        mesh=mesh,
        compiler_params=_SC_PARAMS,
        scratch_types=[
            pltpu.VMEM_SHARED((GP, DH), jnp.float32),
            pltpu.VMEM_SHARED((GP, DH), jnp.float32),
            pltpu.VMEM((1, CH), jnp.int32),
            pltpu.VMEM((CH, DH), jnp.float32),
            pltpu.VMEM((CH, DH), jnp.float32),
        ],
    )
    def k(ha_hbm, hb_hbm, b_hbm, sa_hbm, sb_hbm, cnt_hbm,
          acc, acc2, idx_b, rows, fbuf):
        c = lax.axis_index("c")
        s = lax.axis_index("s")

        @pl.loop(0, CH)
        def _(r):
            @pl.loop(0, DH // 16)
            def _(j):
                fbuf[r, pl.ds(j * 16, 16)] = jnp.zeros((16,), jnp.float32)

        rows_per_sub = GP // NSUB
        pltpu.sync_copy(fbuf.at[pl.ds(0, rows_per_sub)],
                        acc.at[pl.ds(s * rows_per_sub, rows_per_sub)])
        pltpu.sync_copy(fbuf.at[pl.ds(0, rows_per_sub)],
                        acc2.at[pl.ds(s * rows_per_sub, rows_per_sub)])

        @pl.loop(0, CH)
        def _(r):
            @pl.loop(0, DH // 16)
            def _(j):
                fbuf[r, pl.ds(j * 16, 16)] = jnp.full((16,), 1.0, jnp.float32)

        plsc.subcore_barrier()

        def pool_loop(h_hbm, with_counts):
            base = s * (NCHUNKS * CH)

            @pl.loop(0, NCHUNKS)
            def _(kk):
                off = base + kk * CH
                pltpu.sync_copy(b_hbm.at[pl.ds(off, CH)], idx_b.at[0])
                pltpu.sync_copy(h_hbm.at[pl.ds(off, CH)], rows)
                pltpu.sync_copy(rows, acc.at[idx_b.at[0]], add=True)
                if with_counts:
                    pltpu.sync_copy(fbuf, acc2.at[idx_b.at[0]], add=True)

        @pl.when(c == 0)
        def _():
            pool_loop(ha_hbm, True)

        @pl.when(c == 1)
        def _():
            pool_loop(hb_hbm, False)

        plsc.subcore_barrier()

        out_rows = G // NSUB

        def copy_out(o_hbm, a_ref):
            r0 = s * out_rows
            pltpu.sync_copy(a_ref.at[pl.ds(r0, out_rows)],
                            rows.at[pl.ds(0, out_rows)])
            pltpu.sync_copy(rows.at[pl.ds(0, out_rows)],
                            o_hbm.at[pl.ds(r0, out_rows)])

        @pl.when(c == 0)
        def _():
            copy_out(sa_hbm, acc)
            copy_out(cnt_hbm, acc2)

        @pl.when(c == 1)
        def _():
            copy_out(sb_hbm, acc)

    return k(h_a, h_b, batchp)


def _dot_f32x3(a, b):
    return jnp.dot(a.astype(jnp.bfloat16), b.astype(jnp.bfloat16),
                   preferred_element_type=jnp.float32)


def _tc_proj(xp, proj_W, proj_b):
    def body(x_ref, w_ref, b_ref, oa_ref, ob_ref):
        h = _dot_f32x3(x_ref[...], w_ref[...]) + b_ref[...]
        oa_ref[...] = h[:, :DH]
        ob_ref[...] = h[:, DH:]

    return pl.pallas_call(
        body,
        grid=(NBLK,),
        in_specs=[
            pl.BlockSpec((BLK, DIN), lambda i: (i, 0)),
            pl.BlockSpec((DIN, D), lambda i: (0, 0)),
            pl.BlockSpec((1, D), lambda i: (0, 0)),
        ],
        out_specs=[
            pl.BlockSpec((BLK, DH), lambda i: (i, 0)),
            pl.BlockSpec((BLK, DH), lambda i: (i, 0)),
        ],
        out_shape=[jax.ShapeDtypeStruct((NP, DH), jnp.float32),
                   jax.ShapeDtypeStruct((NP, DH), jnp.float32)],
    )(xp, proj_W, proj_b)


def _tc_mlp(h_a, h_b, agg_a, agg_b, W1, b1, W2, b2):
    def body(ha_ref, hb_ref, aa_ref, ab_ref, w1_ref, b1_ref, w2_ref, b2_ref,
             oa_ref, ob_ref):
        z = jnp.concatenate(
            [ha_ref[...] + aa_ref[...], hb_ref[...] + ab_ref[...]], axis=1)
        y = jnp.maximum(_dot_f32x3(z, w1_ref[...]) + b1_ref[...], 0.0)
        o = jnp.maximum(_dot_f32x3(y, w2_ref[...]) + b2_ref[...], 0.0)
        oa_ref[...] = o[:, :DH]
        ob_ref[...] = o[:, DH:]

    row = pl.BlockSpec((BLK, DH), lambda i: (i, 0))
    return pl.pallas_call(
        body,
        grid=(NBLK,),
        in_specs=[
            row, row, row, row,
            pl.BlockSpec((D, D), lambda i: (0, 0)),
            pl.BlockSpec((1, D), lambda i: (0, 0)),
            pl.BlockSpec((D, D), lambda i: (0, 0)),
            pl.BlockSpec((1, D), lambda i: (0, 0)),
        ],
        out_specs=[row, row],
        out_shape=[jax.ShapeDtypeStruct((NP, DH), jnp.float32),
                   jax.ShapeDtypeStruct((NP, DH), jnp.float32)],
    )(h_a, h_b, agg_a, agg_b, W1, b1, W2, b2)


def _tc_head(sums_a, sums_b, counts, shared_W, shared_b, W_heads, b_heads):
    def body(sa_ref, sb_ref, c_ref, w_ref, b_ref, wh_ref, bh_ref, o_ref):
        sums = jnp.concatenate([sa_ref[...], sb_ref[...]], axis=1)
        cnt = jnp.maximum(c_ref[...][:, :1], 1.0)
        pooled = sums / cnt
        s = jnp.maximum(_dot_f32x3(pooled, w_ref[...]) + b_ref[...], 0.0)
        o_ref[...] = _dot_f32x3(s, wh_ref[...]) + bh_ref[...]

    return pl.pallas_call(
        body,
        in_specs=[
            pl.BlockSpec((G, DH), lambda: (0, 0)),
            pl.BlockSpec((G, DH), lambda: (0, 0)),
            pl.BlockSpec((G, DH), lambda: (0, 0)),
            pl.BlockSpec((D, D), lambda: (0, 0)),
            pl.BlockSpec((1, D), lambda: (0, 0)),
            pl.BlockSpec((D, 2), lambda: (0, 0)),
            pl.BlockSpec((1, 2), lambda: (0, 0)),
        ],
        out_specs=pl.BlockSpec((G, 2), lambda: (0, 0)),
        out_shape=jax.ShapeDtypeStruct((G, 2), jnp.float32),
    )(sums_a, sums_b, counts, shared_W, shared_b, W_heads, b_heads)


@jax.jit
def kernel(x, edge_index, batch, proj_W, proj_b,
           conv0_W1, conv0_b1, conv0_W2, conv0_b2,
           conv1_W1, conv1_b1, conv1_W2, conv1_b2,
           conv2_W1, conv2_b1, conv2_W2, conv2_b2,
           shared_W, shared_b, energy_W, energy_b, dipole_W, dipole_b):
    xp = jnp.pad(x, ((0, NP - N), (0, 0)))
    srcp = jnp.concatenate(
        [edge_index[0], jnp.zeros((EPAD - E,), jnp.int32)])
    dstp = jnp.concatenate(
        [edge_index[1], jnp.full((EPAD - E,), N, jnp.int32)])
    batchp = jnp.concatenate([batch, jnp.full((NP - N,), G, jnp.int32)])
    W_heads = jnp.concatenate([energy_W, dipole_W], axis=1)
    b_heads = jnp.concatenate([energy_b, dipole_b]).reshape(1, 2)

    h_a, h_b = _tc_proj(xp, proj_W, proj_b.reshape(1, D))

    convs = [(conv0_W1, conv0_b1, conv0_W2, conv0_b2),
             (conv1_W1, conv1_b1, conv1_W2, conv1_b2),
             (conv2_W1, conv2_b1, conv2_W2, conv2_b2)]
    for (W1, b1, W2, b2) in convs:
        agg_a, agg_b = _sc_edge_agg(h_a, h_b, srcp, dstp)
        h_a, h_b = _tc_mlp(h_a, h_b, agg_a, agg_b,
                           W1, b1.reshape(1, D), W2, b2.reshape(1, D))

    sums_a, sums_b, counts = _sc_pool(h_a, h_b, batchp)
    out = _tc_head(sums_a, sums_b, counts,
                   shared_W, shared_b.reshape(1, D), W_heads, b_heads)
    return (out[:, 0], out[:, 1])


# speedup vs baseline: 6.5943x; 1.7412x over previous
import functools

import jax
import jax.numpy as jnp
from jax import lax
from jax.experimental import pallas as pl
from jax.experimental.pallas import tpu as pltpu
from jax.experimental.pallas import tpu_sc as plsc

_SC_PARAMS = pltpu.CompilerParams(use_tc_tiling_on_sc=False)

N = 50000
E = 800000
DIN = 4
D = 64
DH = 32
G = 512

NSUB = 16
NCORE = 2

NP = 51200
EPAD = 802816
CH = 128
ECHUNKS = EPAD // (NSUB * CH)
NGRP = ECHUNKS // 2
NCHUNKS = NP // (NSUB * CH)
GP = 528

BLK = 1024
NBLK = NP // BLK


def _sc_edge_agg(h_a, h_b, srcp, dstp):
    mesh = plsc.VectorSubcoreMesh(core_axis_name="c", subcore_axis_name="s")

    @functools.partial(
        pl.kernel,
        out_type=[jax.ShapeDtypeStruct((NP, DH), jnp.float32),
                  jax.ShapeDtypeStruct((NP, DH), jnp.float32)],
        mesh=mesh,
        compiler_params=_SC_PARAMS,
        scratch_types=[
            pltpu.VMEM_SHARED((NP, DH), jnp.float32),
            pltpu.VMEM((2, CH), jnp.int32),
            pltpu.VMEM((2, CH), jnp.int32),
            pltpu.VMEM((CH, DH), jnp.float32),
            pltpu.VMEM((CH, DH), jnp.float32),
            pltpu.VMEM((CH, DH), jnp.float32),
            pltpu.SemaphoreType.DMA,
            pltpu.SemaphoreType.DMA,
            pltpu.SemaphoreType.DMA,
            pltpu.SemaphoreType.DMA,
        ],
    )
    def k(ha_hbm, hb_hbm, src_hbm, dst_hbm, oa_hbm, ob_hbm,
          acc, idxs, idxd, rows_a, rows_b, zbuf,
          sem_i0, sem_i1, sem_g0, sem_g1):
        c = lax.axis_index("c")
        s = lax.axis_index("s")

        @pl.loop(0, CH)
        def _(r):
            @pl.loop(0, DH // 16)
            def _(j):
                zbuf[r, pl.ds(j * 16, 16)] = jnp.zeros((16,), jnp.float32)

        @pl.loop(0, NCHUNKS)
        def _(kk):
            pltpu.sync_copy(zbuf, acc.at[pl.ds((s * NCHUNKS + kk) * CH, CH)])

        plsc.subcore_barrier()

        def edge_loop(h_hbm):
            base = s * (ECHUNKS * CH)
            rows = (rows_a, rows_b)
            sem_i = (sem_i0, sem_i1)
            sem_g = (sem_g0, sem_g1)

            def fire_idx(j, p):
                off = base + j * CH
                pltpu.async_copy(src_hbm.at[pl.ds(off, CH)], idxs.at[p],
                                 sem_i[p])
                pltpu.async_copy(dst_hbm.at[pl.ds(off, CH)], idxd.at[p],
                                 sem_i[p])

            def wait_idx(j, p):
                off = base + j * CH
                pltpu.make_async_copy(src_hbm.at[pl.ds(off, CH)], idxs.at[p],
                                      sem_i[p]).wait()
                pltpu.make_async_copy(dst_hbm.at[pl.ds(off, CH)], idxd.at[p],
                                      sem_i[p]).wait()

            def fire_gather(p):
                pltpu.async_copy(h_hbm.at[idxs.at[p]], rows[p], sem_g[p])

            def wait_gather(p):
                pltpu.make_async_copy(h_hbm.at[idxs.at[p]], rows[p],
                                      sem_g[p]).wait()

            def scatter(p):
                pltpu.sync_copy(rows[p], acc.at[idxd.at[p]], add=True)

            pltpu.sync_copy(src_hbm.at[pl.ds(base, CH)], idxs.at[0])
            pltpu.sync_copy(dst_hbm.at[pl.ds(base, CH)], idxd.at[0])
            fire_gather(0)
            fire_idx(1, 1)

            @pl.loop(0, ECHUNKS // 2)
            def _(kk):
                j = 2 * kk
                wait_gather(0)
                wait_idx(j + 1, 1)
                fire_gather(1)
                scatter(0)

                @pl.when(j + 2 < ECHUNKS)
                def _():
                    fire_idx(j + 2, 0)

                wait_gather(1)

                @pl.when(j + 2 < ECHUNKS)
                def _():
                    wait_idx(j + 2, 0)
                    fire_gather(0)

                scatter(1)

                @pl.when(j + 3 < ECHUNKS)
                def _():
                    fire_idx(j + 3, 1)

        @pl.when(c == 0)
        def _():
            edge_loop(ha_hbm)

        @pl.when(c == 1)
        def _():
            edge_loop(hb_hbm)

        plsc.subcore_barrier()

        def copy_out(o_hbm):
            @pl.loop(0, NCHUNKS)
            def _(kk):
                r0 = (s * NCHUNKS + kk) * CH
                pltpu.sync_copy(acc.at[pl.ds(r0, CH)], o_hbm.at[pl.ds(r0, CH)])

        @pl.when(c == 0)
        def _():
            copy_out(oa_hbm)

        @pl.when(c == 1)
        def _():
            copy_out(ob_hbm)

    return k(h_a, h_b, srcp, dstp)


def _sc_pool(h_a, h_b, batchp):
    mesh = plsc.VectorSubcoreMesh(core_axis_name="c", subcore_axis_name="s")

    @functools.partial(
        pl.kernel,
        out_type=[jax.ShapeDtypeStruct((G, DH), jnp.float32),
                  jax.ShapeDtypeStruct((G, DH), jnp.float32),
                  jax.ShapeDtypeStruct((G, DH), jnp.float32)],
        mesh=mesh,
        compiler_params=_SC_PARAMS,
        scratch_types=[
            pltpu.VMEM_SHARED((GP, DH), jnp.float32),
            pltpu.VMEM_SHARED((GP, DH), jnp.float32),
            pltpu.VMEM((1, CH), jnp.int32),
            pltpu.VMEM((CH, DH), jnp.float32),
            pltpu.VMEM((CH, DH), jnp.float32),
        ],
    )
    def k(ha_hbm, hb_hbm, b_hbm, sa_hbm, sb_hbm, cnt_hbm,
          acc, acc2, idx_b, rows, fbuf):
        c = lax.axis_index("c")
        s = lax.axis_index("s")

        @pl.loop(0, CH)
        def _(r):
            @pl.loop(0, DH // 16)
            def _(j):
                fbuf[r, pl.ds(j * 16, 16)] = jnp.zeros((16,), jnp.float32)

        rows_per_sub = GP // NSUB
        pltpu.sync_copy(fbuf.at[pl.ds(0, rows_per_sub)],
                        acc.at[pl.ds(s * rows_per_sub, rows_per_sub)])
        pltpu.sync_copy(fbuf.at[pl.ds(0, rows_per_sub)],
                        acc2.at[pl.ds(s * rows_per_sub, rows_per_sub)])

        @pl.loop(0, CH)
        def _(r):
            @pl.loop(0, DH // 16)
            def _(j):
                fbuf[r, pl.ds(j * 16, 16)] = jnp.full((16,), 1.0, jnp.float32)

        plsc.subcore_barrier()

        def pool_loop(h_hbm, with_counts):
            base = s * (NCHUNKS * CH)

            @pl.loop(0, NCHUNKS)
            def _(kk):
                off = base + kk * CH
                pltpu.sync_copy(b_hbm.at[pl.ds(off, CH)], idx_b.at[0])
                pltpu.sync_copy(h_hbm.at[pl.ds(off, CH)], rows)
                pltpu.sync_copy(rows, acc.at[idx_b.at[0]], add=True)
                if with_counts:
                    pltpu.sync_copy(fbuf, acc2.at[idx_b.at[0]], add=True)

        @pl.when(c == 0)
        def _():
            pool_loop(ha_hbm, True)

        @pl.when(c == 1)
        def _():
            pool_loop(hb_hbm, False)

        plsc.subcore_barrier()

        out_rows = G // NSUB

        def copy_out(o_hbm, a_ref):
            r0 = s * out_rows
            pltpu.sync_copy(a_ref.at[pl.ds(r0, out_rows)],
                            rows.at[pl.ds(0, out_rows)])
            pltpu.sync_copy(rows.at[pl.ds(0, out_rows)],
                            o_hbm.at[pl.ds(r0, out_rows)])

        @pl.when(c == 0)
        def _():
            copy_out(sa_hbm, acc)
            copy_out(cnt_hbm, acc2)

        @pl.when(c == 1)
        def _():
            copy_out(sb_hbm, acc)

    return k(h_a, h_b, batchp)


def _dot_f32x3(a, b):
    return jnp.dot(a.astype(jnp.bfloat16), b.astype(jnp.bfloat16),
                   preferred_element_type=jnp.float32)


def _tc_proj(xp, proj_W, proj_b):
    def body(x_ref, w_ref, b_ref, oa_ref, ob_ref):
        h = _dot_f32x3(x_ref[...], w_ref[...]) + b_ref[...]
        oa_ref[...] = h[:, :DH]
        ob_ref[...] = h[:, DH:]

    return pl.pallas_call(
        body,
        grid=(NBLK,),
        in_specs=[
            pl.BlockSpec((BLK, DIN), lambda i: (i, 0)),
            pl.BlockSpec((DIN, D), lambda i: (0, 0)),
            pl.BlockSpec((1, D), lambda i: (0, 0)),
        ],
        out_specs=[
            pl.BlockSpec((BLK, DH), lambda i: (i, 0)),
            pl.BlockSpec((BLK, DH), lambda i: (i, 0)),
        ],
        out_shape=[jax.ShapeDtypeStruct((NP, DH), jnp.float32),
                   jax.ShapeDtypeStruct((NP, DH), jnp.float32)],
    )(xp, proj_W, proj_b)


def _tc_mlp(h_a, h_b, agg_a, agg_b, W1, b1, W2, b2):
    def body(ha_ref, hb_ref, aa_ref, ab_ref, w1_ref, b1_ref, w2_ref, b2_ref,
             oa_ref, ob_ref):
        z = jnp.concatenate(
            [ha_ref[...] + aa_ref[...], hb_ref[...] + ab_ref[...]], axis=1)
        y = jnp.maximum(_dot_f32x3(z, w1_ref[...]) + b1_ref[...], 0.0)
        o = jnp.maximum(_dot_f32x3(y, w2_ref[...]) + b2_ref[...], 0.0)
        oa_ref[...] = o[:, :DH]
        ob_ref[...] = o[:, DH:]

    row = pl.BlockSpec((BLK, DH), lambda i: (i, 0))
    return pl.pallas_call(
        body,
        grid=(NBLK,),
        in_specs=[
            row, row, row, row,
            pl.BlockSpec((D, D), lambda i: (0, 0)),
            pl.BlockSpec((1, D), lambda i: (0, 0)),
            pl.BlockSpec((D, D), lambda i: (0, 0)),
            pl.BlockSpec((1, D), lambda i: (0, 0)),
        ],
        out_specs=[row, row],
        out_shape=[jax.ShapeDtypeStruct((NP, DH), jnp.float32),
                   jax.ShapeDtypeStruct((NP, DH), jnp.float32)],
    )(h_a, h_b, agg_a, agg_b, W1, b1, W2, b2)


def _tc_head(sums_a, sums_b, counts, shared_W, shared_b, W_heads, b_heads):
    def body(sa_ref, sb_ref, c_ref, w_ref, b_ref, wh_ref, bh_ref, o_ref):
        sums = jnp.concatenate([sa_ref[...], sb_ref[...]], axis=1)
        cnt = jnp.maximum(c_ref[...][:, :1], 1.0)
        pooled = sums / cnt
        s = jnp.maximum(_dot_f32x3(pooled, w_ref[...]) + b_ref[...], 0.0)
        o_ref[...] = _dot_f32x3(s, wh_ref[...]) + bh_ref[...]

    return pl.pallas_call(
        body,
        in_specs=[
            pl.BlockSpec((G, DH), lambda: (0, 0)),
            pl.BlockSpec((G, DH), lambda: (0, 0)),
            pl.BlockSpec((G, DH), lambda: (0, 0)),
            pl.BlockSpec((D, D), lambda: (0, 0)),
            pl.BlockSpec((1, D), lambda: (0, 0)),
            pl.BlockSpec((D, 2), lambda: (0, 0)),
            pl.BlockSpec((1, 2), lambda: (0, 0)),
        ],
        out_specs=pl.BlockSpec((G, 2), lambda: (0, 0)),
        out_shape=jax.ShapeDtypeStruct((G, 2), jnp.float32),
    )(sums_a, sums_b, counts, shared_W, shared_b, W_heads, b_heads)


@jax.jit
def kernel(x, edge_index, batch, proj_W, proj_b,
           conv0_W1, conv0_b1, conv0_W2, conv0_b2,
           conv1_W1, conv1_b1, conv1_W2, conv1_b2,
           conv2_W1, conv2_b1, conv2_W2, conv2_b2,
           shared_W, shared_b, energy_W, energy_b, dipole_W, dipole_b):
    xp = jnp.pad(x, ((0, NP - N), (0, 0)))
    srcp = jnp.concatenate(
        [edge_index[0], jnp.zeros((EPAD - E,), jnp.int32)])
    dstp = jnp.concatenate(
        [edge_index[1], jnp.full((EPAD - E,), N, jnp.int32)])
    batchp = jnp.concatenate([batch, jnp.full((NP - N,), G, jnp.int32)])
    W_heads = jnp.concatenate([energy_W, dipole_W], axis=1)
    b_heads = jnp.concatenate([energy_b, dipole_b]).reshape(1, 2)

    h_a, h_b = _tc_proj(xp, proj_W, proj_b.reshape(1, D))

    convs = [(conv0_W1, conv0_b1, conv0_W2, conv0_b2),
             (conv1_W1, conv1_b1, conv1_W2, conv1_b2),
             (conv2_W1, conv2_b1, conv2_W2, conv2_b2)]
    for (W1, b1, W2, b2) in convs:
        agg_a, agg_b = _sc_edge_agg(h_a, h_b, srcp, dstp)
        h_a, h_b = _tc_mlp(h_a, h_b, agg_a, agg_b,
                           W1, b1.reshape(1, D), W2, b2.reshape(1, D))

    sums_a, sums_b, counts = _sc_pool(h_a, h_b, batchp)
    out = _tc_head(sums_a, sums_b, counts,
                   shared_W, shared_b.reshape(1, D), W_heads, b_heads)
    return (out[:, 0], out[:, 1])


# speedup vs baseline: 7.0129x; 1.0635x over previous
import functools

import jax
import jax.numpy as jnp
from jax import lax
from jax.experimental import pallas as pl
from jax.experimental.pallas import tpu as pltpu
from jax.experimental.pallas import tpu_sc as plsc

_SC_PARAMS = pltpu.CompilerParams(use_tc_tiling_on_sc=False)

N = 50000
E = 800000
DIN = 4
D = 64
DH = 32
G = 512

NSUB = 16
NCORE = 2

NP = 51200
EPAD = 802816
CH = 128
ECHUNKS = EPAD // (NSUB * CH)
NGRP = ECHUNKS // 2
NCHUNKS = NP // (NSUB * CH)
GP = 528

BLK = 1024
NBLK = NP // BLK


def _sc_edge_agg(h_a, h_b, srcp, dstp):
    mesh = plsc.VectorSubcoreMesh(core_axis_name="c", subcore_axis_name="s")

    @functools.partial(
        pl.kernel,
        out_type=[jax.ShapeDtypeStruct((NP, DH), jnp.float32),
                  jax.ShapeDtypeStruct((NP, DH), jnp.float32)],
        mesh=mesh,
        compiler_params=_SC_PARAMS,
        scratch_types=[
            pltpu.VMEM_SHARED((NP, DH), jnp.float32),
            pltpu.VMEM((4, CH), jnp.int32),
            pltpu.VMEM((4, CH), jnp.int32),
            pltpu.VMEM((CH, DH), jnp.float32),
            pltpu.VMEM((CH, DH), jnp.float32),
            pltpu.VMEM((CH, DH), jnp.float32),
            pltpu.VMEM((CH, DH), jnp.float32),
            pltpu.VMEM((CH, DH), jnp.float32),
            pltpu.SemaphoreType.DMA,
            pltpu.SemaphoreType.DMA,
            pltpu.SemaphoreType.DMA,
            pltpu.SemaphoreType.DMA,
            pltpu.SemaphoreType.DMA,
            pltpu.SemaphoreType.DMA,
            pltpu.SemaphoreType.DMA,
            pltpu.SemaphoreType.DMA,
        ],
    )
    def k(ha_hbm, hb_hbm, src_hbm, dst_hbm, oa_hbm, ob_hbm,
          acc, idxs, idxd, rows0, rows1, rows2, rows3, zbuf,
          si0, si1, si2, si3, sg0, sg1, sg2, sg3):
        c = lax.axis_index("c")
        s = lax.axis_index("s")

        @pl.loop(0, CH)
        def _(r):
            @pl.loop(0, DH // 16)
            def _(j):
                zbuf[r, pl.ds(j * 16, 16)] = jnp.zeros((16,), jnp.float32)

        @pl.loop(0, NCHUNKS)
        def _(kk):
            pltpu.sync_copy(zbuf, acc.at[pl.ds((s * NCHUNKS + kk) * CH, CH)])

        plsc.subcore_barrier()

        rows = (rows0, rows1, rows2, rows3)
        sem_i = (si0, si1, si2, si3)
        sem_g = (sg0, sg1, sg2, sg3)

        def edge_loop(h_hbm):
            base = s * (ECHUNKS * CH)

            def fire_idx(j, b):
                off = base + j * CH
                pltpu.async_copy(src_hbm.at[pl.ds(off, CH)], idxs.at[b],
                                 sem_i[b])
                pltpu.async_copy(dst_hbm.at[pl.ds(off, CH)], idxd.at[b],
                                 sem_i[b])

            def wait_idx(j, b):
                off = base + j * CH
                pltpu.make_async_copy(src_hbm.at[pl.ds(off, CH)], idxs.at[b],
                                      sem_i[b]).wait()
                pltpu.make_async_copy(dst_hbm.at[pl.ds(off, CH)], idxd.at[b],
                                      sem_i[b]).wait()

            def fire_gather(b):
                pltpu.async_copy(h_hbm.at[idxs.at[b]], rows[b], sem_g[b])

            def wait_gather(b):
                pltpu.make_async_copy(h_hbm.at[idxs.at[b]], rows[b],
                                      sem_g[b]).wait()

            def scatter(b):
                pltpu.sync_copy(rows[b], acc.at[idxd.at[b]], add=True)

            fire_idx(0, 0)
            fire_idx(1, 1)
            fire_idx(2, 2)
            wait_idx(0, 0)
            fire_gather(0)
            wait_idx(1, 1)
            fire_gather(1)

            @pl.loop(0, ECHUNKS // 4)
            def _(it):
                for m in range(4):
                    b = m
                    k_ = 4 * it + m
                    wait_gather(b)

                    @pl.when(k_ + 2 < ECHUNKS)
                    def _():
                        wait_idx(k_ + 2, (b + 2) % 4)
                        fire_gather((b + 2) % 4)

                    scatter(b)

                    @pl.when(k_ + 3 < ECHUNKS)
                    def _():
                        fire_idx(k_ + 3, (b + 3) % 4)

        @pl.when(c == 0)
        def _():
            edge_loop(ha_hbm)

        @pl.when(c == 1)
        def _():
            edge_loop(hb_hbm)

        plsc.subcore_barrier()

        def copy_out(o_hbm):
            @pl.loop(0, NCHUNKS)
            def _(kk):
                r0 = (s * NCHUNKS + kk) * CH
                pltpu.sync_copy(acc.at[pl.ds(r0, CH)], o_hbm.at[pl.ds(r0, CH)])

        @pl.when(c == 0)
        def _():
            copy_out(oa_hbm)

        @pl.when(c == 1)
        def _():
            copy_out(ob_hbm)

    return k(h_a, h_b, srcp, dstp)


def _sc_pool(h_a, h_b, batchp):
    mesh = plsc.VectorSubcoreMesh(core_axis_name="c", subcore_axis_name="s")

    @functools.partial(
        pl.kernel,
        out_type=[jax.ShapeDtypeStruct((G, DH), jnp.float32),
                  jax.ShapeDtypeStruct((G, DH), jnp.float32),
                  jax.ShapeDtypeStruct((G, DH), jnp.float32)],
        mesh=mesh,
        compiler_params=_SC_PARAMS,
        scratch_types=[
            pltpu.VMEM_SHARED((GP, DH), jnp.float32),
            pltpu.VMEM_SHARED((GP, DH), jnp.float32),
            pltpu.VMEM((1, CH), jnp.int32),
            pltpu.VMEM((CH, DH), jnp.float32),
            pltpu.VMEM((CH, DH), jnp.float32),
        ],
    )
    def k(ha_hbm, hb_hbm, b_hbm, sa_hbm, sb_hbm, cnt_hbm,
          acc, acc2, idx_b, rows, fbuf):
        c = lax.axis_index("c")
        s = lax.axis_index("s")

        @pl.loop(0, CH)
        def _(r):
            @pl.loop(0, DH // 16)
            def _(j):
                fbuf[r, pl.ds(j * 16, 16)] = jnp.zeros((16,), jnp.float32)

        rows_per_sub = GP // NSUB
        pltpu.sync_copy(fbuf.at[pl.ds(0, rows_per_sub)],
                        acc.at[pl.ds(s * rows_per_sub, rows_per_sub)])
        pltpu.sync_copy(fbuf.at[pl.ds(0, rows_per_sub)],
                        acc2.at[pl.ds(s * rows_per_sub, rows_per_sub)])

        @pl.loop(0, CH)
        def _(r):
            @pl.loop(0, DH // 16)
            def _(j):
                fbuf[r, pl.ds(j * 16, 16)] = jnp.full((16,), 1.0, jnp.float32)

        plsc.subcore_barrier()

        def pool_loop(h_hbm, with_counts):
            base = s * (NCHUNKS * CH)

            @pl.loop(0, NCHUNKS)
            def _(kk):
                off = base + kk * CH
                pltpu.sync_copy(b_hbm.at[pl.ds(off, CH)], idx_b.at[0])
                pltpu.sync_copy(h_hbm.at[pl.ds(off, CH)], rows)
                pltpu.sync_copy(rows, acc.at[idx_b.at[0]], add=True)
                if with_counts:
                    pltpu.sync_copy(fbuf, acc2.at[idx_b.at[0]], add=True)

        @pl.when(c == 0)
        def _():
            pool_loop(ha_hbm, True)

        @pl.when(c == 1)
        def _():
            pool_loop(hb_hbm, False)

        plsc.subcore_barrier()

        out_rows = G // NSUB

        def copy_out(o_hbm, a_ref):
            r0 = s * out_rows
            pltpu.sync_copy(a_ref.at[pl.ds(r0, out_rows)],
                            rows.at[pl.ds(0, out_rows)])
            pltpu.sync_copy(rows.at[pl.ds(0, out_rows)],
                            o_hbm.at[pl.ds(r0, out_rows)])

        @pl.when(c == 0)
        def _():
            copy_out(sa_hbm, acc)
            copy_out(cnt_hbm, acc2)

        @pl.when(c == 1)
        def _():
            copy_out(sb_hbm, acc)

    return k(h_a, h_b, batchp)


def _dot_f32x3(a, b):
    return jnp.dot(a.astype(jnp.bfloat16), b.astype(jnp.bfloat16),
                   preferred_element_type=jnp.float32)


def _tc_proj(xp, proj_W, proj_b):
    def body(x_ref, w_ref, b_ref, oa_ref, ob_ref):
        h = _dot_f32x3(x_ref[...], w_ref[...]) + b_ref[...]
        oa_ref[...] = h[:, :DH]
        ob_ref[...] = h[:, DH:]

    return pl.pallas_call(
        body,
        grid=(NBLK,),
        in_specs=[
            pl.BlockSpec((BLK, DIN), lambda i: (i, 0)),
            pl.BlockSpec((DIN, D), lambda i: (0, 0)),
            pl.BlockSpec((1, D), lambda i: (0, 0)),
        ],
        out_specs=[
            pl.BlockSpec((BLK, DH), lambda i: (i, 0)),
            pl.BlockSpec((BLK, DH), lambda i: (i, 0)),
        ],
        out_shape=[jax.ShapeDtypeStruct((NP, DH), jnp.float32),
                   jax.ShapeDtypeStruct((NP, DH), jnp.float32)],
    )(xp, proj_W, proj_b)


def _tc_mlp(h_a, h_b, agg_a, agg_b, W1, b1, W2, b2):
    def body(ha_ref, hb_ref, aa_ref, ab_ref, w1_ref, b1_ref, w2_ref, b2_ref,
             oa_ref, ob_ref):
        z = jnp.concatenate(
            [ha_ref[...] + aa_ref[...], hb_ref[...] + ab_ref[...]], axis=1)
        y = jnp.maximum(_dot_f32x3(z, w1_ref[...]) + b1_ref[...], 0.0)
        o = jnp.maximum(_dot_f32x3(y, w2_ref[...]) + b2_ref[...], 0.0)
        oa_ref[...] = o[:, :DH]
        ob_ref[...] = o[:, DH:]

    row = pl.BlockSpec((BLK, DH), lambda i: (i, 0))
    return pl.pallas_call(
        body,
        grid=(NBLK,),
        in_specs=[
            row, row, row, row,
            pl.BlockSpec((D, D), lambda i: (0, 0)),
            pl.BlockSpec((1, D), lambda i: (0, 0)),
            pl.BlockSpec((D, D), lambda i: (0, 0)),
            pl.BlockSpec((1, D), lambda i: (0, 0)),
        ],
        out_specs=[row, row],
        out_shape=[jax.ShapeDtypeStruct((NP, DH), jnp.float32),
                   jax.ShapeDtypeStruct((NP, DH), jnp.float32)],
    )(h_a, h_b, agg_a, agg_b, W1, b1, W2, b2)


def _tc_head(sums_a, sums_b, counts, shared_W, shared_b, W_heads, b_heads):
    def body(sa_ref, sb_ref, c_ref, w_ref, b_ref, wh_ref, bh_ref, o_ref):
        sums = jnp.concatenate([sa_ref[...], sb_ref[...]], axis=1)
        cnt = jnp.maximum(c_ref[...][:, :1], 1.0)
        pooled = sums / cnt
        s = jnp.maximum(_dot_f32x3(pooled, w_ref[...]) + b_ref[...], 0.0)
        o_ref[...] = _dot_f32x3(s, wh_ref[...]) + bh_ref[...]

    return pl.pallas_call(
        body,
        in_specs=[
            pl.BlockSpec((G, DH), lambda: (0, 0)),
            pl.BlockSpec((G, DH), lambda: (0, 0)),
            pl.BlockSpec((G, DH), lambda: (0, 0)),
            pl.BlockSpec((D, D), lambda: (0, 0)),
            pl.BlockSpec((1, D), lambda: (0, 0)),
            pl.BlockSpec((D, 2), lambda: (0, 0)),
            pl.BlockSpec((1, 2), lambda: (0, 0)),
        ],
        out_specs=pl.BlockSpec((G, 2), lambda: (0, 0)),
        out_shape=jax.ShapeDtypeStruct((G, 2), jnp.float32),
    )(sums_a, sums_b, counts, shared_W, shared_b, W_heads, b_heads)


@jax.jit
def kernel(x, edge_index, batch, proj_W, proj_b,
           conv0_W1, conv0_b1, conv0_W2, conv0_b2,
           conv1_W1, conv1_b1, conv1_W2, conv1_b2,
           conv2_W1, conv2_b1, conv2_W2, conv2_b2,
           shared_W, shared_b, energy_W, energy_b, dipole_W, dipole_b):
    xp = jnp.pad(x, ((0, NP - N), (0, 0)))
    srcp = jnp.concatenate(
        [edge_index[0], jnp.zeros((EPAD - E,), jnp.int32)])
    dstp = jnp.concatenate(
        [edge_index[1], jnp.full((EPAD - E,), N, jnp.int32)])
    batchp = jnp.concatenate([batch, jnp.full((NP - N,), G, jnp.int32)])
    W_heads = jnp.concatenate([energy_W, dipole_W], axis=1)
    b_heads = jnp.concatenate([energy_b, dipole_b]).reshape(1, 2)

    h_a, h_b = _tc_proj(xp, proj_W, proj_b.reshape(1, D))

    convs = [(conv0_W1, conv0_b1, conv0_W2, conv0_b2),
             (conv1_W1, conv1_b1, conv1_W2, conv1_b2),
             (conv2_W1, conv2_b1, conv2_W2, conv2_b2)]
    for (W1, b1, W2, b2) in convs:
        agg_a, agg_b = _sc_edge_agg(h_a, h_b, srcp, dstp)
        h_a, h_b = _tc_mlp(h_a, h_b, agg_a, agg_b,
                           W1, b1.reshape(1, D), W2, b2.reshape(1, D))

    sums_a, sums_b, counts = _sc_pool(h_a, h_b, batchp)
    out = _tc_head(sums_a, sums_b, counts,
                   shared_W, shared_b.reshape(1, D), W_heads, b_heads)
    return (out[:, 0], out[:, 1])


# speedup vs baseline: 8.0062x; 1.1416x over previous
import functools

import jax
import jax.numpy as jnp
from jax import lax
from jax.experimental import pallas as pl
from jax.experimental.pallas import tpu as pltpu
from jax.experimental.pallas import tpu_sc as plsc

_SC_PARAMS = pltpu.CompilerParams(use_tc_tiling_on_sc=False)

N = 50000
E = 800000
DIN = 4
D = 64
DH = 32
G = 512

NSUB = 16
NCORE = 2

NP = 51200
EPAD = 802816
CH = 128
ECHUNKS = EPAD // (NSUB * CH)
NGRP = ECHUNKS // 2
NCHUNKS = NP // (NSUB * CH)
GP = 528

BLK = 1024
NBLK = NP // BLK


def _sc_edge_agg(h_a, h_b, srcp, dstp):
    mesh = plsc.VectorSubcoreMesh(core_axis_name="c", subcore_axis_name="s")

    @functools.partial(
        pl.kernel,
        out_type=[jax.ShapeDtypeStruct((NP, DH), jnp.float32),
                  jax.ShapeDtypeStruct((NP, DH), jnp.float32)],
        mesh=mesh,
        compiler_params=_SC_PARAMS,
        scratch_types=[
            pltpu.VMEM_SHARED((NP, DH), jnp.float32),
            pltpu.VMEM((8, CH), jnp.int32),
            pltpu.VMEM((8, CH), jnp.int32),
            pltpu.VMEM((CH, DH), jnp.float32),
            pltpu.VMEM((CH, DH), jnp.float32),
            pltpu.VMEM((CH, DH), jnp.float32),
            pltpu.VMEM((CH, DH), jnp.float32),
            pltpu.VMEM((CH, DH), jnp.float32),
        ] + [pltpu.SemaphoreType.DMA] * 20,
    )
    def k(ha_hbm, hb_hbm, src_hbm, dst_hbm, oa_hbm, ob_hbm,
          acc, idxs, idxd, rows0, rows1, rows2, rows3, zbuf, *sems):
        c = lax.axis_index("c")
        s = lax.axis_index("s")
        sem_i = sems[0:8]
        sem_g = sems[8:12]
        sem_s = sems[12:20]

        @pl.loop(0, CH)
        def _(r):
            @pl.loop(0, DH // 16)
            def _(j):
                zbuf[r, pl.ds(j * 16, 16)] = jnp.zeros((16,), jnp.float32)

        @pl.loop(0, NCHUNKS)
        def _(kk):
            pltpu.sync_copy(zbuf, acc.at[pl.ds((s * NCHUNKS + kk) * CH, CH)])

        plsc.subcore_barrier()

        rows = (rows0, rows1, rows2, rows3)

        def edge_loop(h_hbm):
            base = s * (ECHUNKS * CH)

            def fire_idx(j, b):
                off = base + j * CH
                pltpu.async_copy(src_hbm.at[pl.ds(off, CH)], idxs.at[b],
                                 sem_i[b])
                pltpu.async_copy(dst_hbm.at[pl.ds(off, CH)], idxd.at[b],
                                 sem_i[b])

            def wait_idx(j, b):
                off = base + j * CH
                pltpu.make_async_copy(src_hbm.at[pl.ds(off, CH)], idxs.at[b],
                                      sem_i[b]).wait()
                pltpu.make_async_copy(dst_hbm.at[pl.ds(off, CH)], idxd.at[b],
                                      sem_i[b]).wait()

            def fire_gather(bi, br):
                pltpu.async_copy(h_hbm.at[idxs.at[bi]], rows[br], sem_g[br])

            def wait_gather(bi, br):
                pltpu.make_async_copy(h_hbm.at[idxs.at[bi]], rows[br],
                                      sem_g[br]).wait()

            fire_idx(0, 0)
            fire_idx(1, 1)
            fire_idx(2, 2)
            wait_idx(0, 0)
            fire_gather(0, 0)
            wait_idx(1, 1)
            fire_gather(1, 1)

            @pl.loop(0, ECHUNKS // 8)
            def _(it):
                handles = [None] * 8
                for m in range(8):
                    k_ = 8 * it + m
                    bi = m % 8
                    br = m % 4
                    wait_gather(bi, br)

                    if m >= 2:
                        handles[m - 2].wait()

                    @pl.when(k_ + 2 < ECHUNKS)
                    def _():
                        wait_idx(k_ + 2, (bi + 2) % 8)
                        fire_gather((bi + 2) % 8, (br + 2) % 4)

                    handles[m] = pltpu.async_copy(
                        rows[br], acc.at[idxd.at[bi]], sem_s[m], add=True)

                    @pl.when(k_ + 3 < ECHUNKS)
                    def _():
                        fire_idx(k_ + 3, (bi + 3) % 8)

                handles[6].wait()
                handles[7].wait()

        @pl.when(c == 0)
        def _():
            edge_loop(ha_hbm)

        @pl.when(c == 1)
        def _():
            edge_loop(hb_hbm)

        plsc.subcore_barrier()

        def copy_out(o_hbm):
            @pl.loop(0, NCHUNKS)
            def _(kk):
                r0 = (s * NCHUNKS + kk) * CH
                pltpu.sync_copy(acc.at[pl.ds(r0, CH)], o_hbm.at[pl.ds(r0, CH)])

        @pl.when(c == 0)
        def _():
            copy_out(oa_hbm)

        @pl.when(c == 1)
        def _():
            copy_out(ob_hbm)

    return k(h_a, h_b, srcp, dstp)


def _sc_pool(h_a, h_b, batchp):
    mesh = plsc.VectorSubcoreMesh(core_axis_name="c", subcore_axis_name="s")

    @functools.partial(
        pl.kernel,
        out_type=[jax.ShapeDtypeStruct((G, DH), jnp.float32),
                  jax.ShapeDtypeStruct((G, DH), jnp.float32),
                  jax.ShapeDtypeStruct((G, DH), jnp.float32)],
        mesh=mesh,
        compiler_params=_SC_PARAMS,
        scratch_types=[
            pltpu.VMEM_SHARED((GP, DH), jnp.float32),
            pltpu.VMEM_SHARED((GP, DH), jnp.float32),
            pltpu.VMEM((1, CH), jnp.int32),
            pltpu.VMEM((CH, DH), jnp.float32),
            pltpu.VMEM((CH, DH), jnp.float32),
        ],
    )
    def k(ha_hbm, hb_hbm, b_hbm, sa_hbm, sb_hbm, cnt_hbm,
          acc, acc2, idx_b, rows, fbuf):
        c = lax.axis_index("c")
        s = lax.axis_index("s")

        @pl.loop(0, CH)
        def _(r):
            @pl.loop(0, DH // 16)
            def _(j):
                fbuf[r, pl.ds(j * 16, 16)] = jnp.zeros((16,), jnp.float32)

        rows_per_sub = GP // NSUB
        pltpu.sync_copy(fbuf.at[pl.ds(0, rows_per_sub)],
                        acc.at[pl.ds(s * rows_per_sub, rows_per_sub)])
        pltpu.sync_copy(fbuf.at[pl.ds(0, rows_per_sub)],
                        acc2.at[pl.ds(s * rows_per_sub, rows_per_sub)])

        @pl.loop(0, CH)
        def _(r):
            @pl.loop(0, DH // 16)
            def _(j):
                fbuf[r, pl.ds(j * 16, 16)] = jnp.full((16,), 1.0, jnp.float32)

        plsc.subcore_barrier()

        def pool_loop(h_hbm, with_counts):
            base = s * (NCHUNKS * CH)

            @pl.loop(0, NCHUNKS)
            def _(kk):
                off = base + kk * CH
                pltpu.sync_copy(b_hbm.at[pl.ds(off, CH)], idx_b.at[0])
                pltpu.sync_copy(h_hbm.at[pl.ds(off, CH)], rows)
                pltpu.sync_copy(rows, acc.at[idx_b.at[0]], add=True)
                if with_counts:
                    pltpu.sync_copy(fbuf, acc2.at[idx_b.at[0]], add=True)

        @pl.when(c == 0)
        def _():
            pool_loop(ha_hbm, True)

        @pl.when(c == 1)
        def _():
            pool_loop(hb_hbm, False)

        plsc.subcore_barrier()

        out_rows = G // NSUB

        def copy_out(o_hbm, a_ref):
            r0 = s * out_rows
            pltpu.sync_copy(a_ref.at[pl.ds(r0, out_rows)],
                            rows.at[pl.ds(0, out_rows)])
            pltpu.sync_copy(rows.at[pl.ds(0, out_rows)],
                            o_hbm.at[pl.ds(r0, out_rows)])

        @pl.when(c == 0)
        def _():
            copy_out(sa_hbm, acc)
            copy_out(cnt_hbm, acc2)

        @pl.when(c == 1)
        def _():
            copy_out(sb_hbm, acc)

    return k(h_a, h_b, batchp)


def _dot_f32x3(a, b):
    return jnp.dot(a.astype(jnp.bfloat16), b.astype(jnp.bfloat16),
                   preferred_element_type=jnp.float32)


def _tc_proj(xp, proj_W, proj_b):
    def body(x_ref, w_ref, b_ref, oa_ref, ob_ref):
        h = _dot_f32x3(x_ref[...], w_ref[...]) + b_ref[...]
        oa_ref[...] = h[:, :DH]
        ob_ref[...] = h[:, DH:]

    return pl.pallas_call(
        body,
        grid=(NBLK,),
        in_specs=[
            pl.BlockSpec((BLK, DIN), lambda i: (i, 0)),
            pl.BlockSpec((DIN, D), lambda i: (0, 0)),
            pl.BlockSpec((1, D), lambda i: (0, 0)),
        ],
        out_specs=[
            pl.BlockSpec((BLK, DH), lambda i: (i, 0)),
            pl.BlockSpec((BLK, DH), lambda i: (i, 0)),
        ],
        out_shape=[jax.ShapeDtypeStruct((NP, DH), jnp.float32),
                   jax.ShapeDtypeStruct((NP, DH), jnp.float32)],
    )(xp, proj_W, proj_b)


def _tc_mlp(h_a, h_b, agg_a, agg_b, W1, b1, W2, b2):
    def body(ha_ref, hb_ref, aa_ref, ab_ref, w1_ref, b1_ref, w2_ref, b2_ref,
             oa_ref, ob_ref):
        z = jnp.concatenate(
            [ha_ref[...] + aa_ref[...], hb_ref[...] + ab_ref[...]], axis=1)
        y = jnp.maximum(_dot_f32x3(z, w1_ref[...]) + b1_ref[...], 0.0)
        o = jnp.maximum(_dot_f32x3(y, w2_ref[...]) + b2_ref[...], 0.0)
        oa_ref[...] = o[:, :DH]
        ob_ref[...] = o[:, DH:]

    row = pl.BlockSpec((BLK, DH), lambda i: (i, 0))
    return pl.pallas_call(
        body,
        grid=(NBLK,),
        in_specs=[
            row, row, row, row,
            pl.BlockSpec((D, D), lambda i: (0, 0)),
            pl.BlockSpec((1, D), lambda i: (0, 0)),
            pl.BlockSpec((D, D), lambda i: (0, 0)),
            pl.BlockSpec((1, D), lambda i: (0, 0)),
        ],
        out_specs=[row, row],
        out_shape=[jax.ShapeDtypeStruct((NP, DH), jnp.float32),
                   jax.ShapeDtypeStruct((NP, DH), jnp.float32)],
    )(h_a, h_b, agg_a, agg_b, W1, b1, W2, b2)


def _tc_head(sums_a, sums_b, counts, shared_W, shared_b, W_heads, b_heads):
    def body(sa_ref, sb_ref, c_ref, w_ref, b_ref, wh_ref, bh_ref, o_ref):
        sums = jnp.concatenate([sa_ref[...], sb_ref[...]], axis=1)
        cnt = jnp.maximum(c_ref[...][:, :1], 1.0)
        pooled = sums / cnt
        s = jnp.maximum(_dot_f32x3(pooled, w_ref[...]) + b_ref[...], 0.0)
        o_ref[...] = _dot_f32x3(s, wh_ref[...]) + bh_ref[...]

    return pl.pallas_call(
        body,
        in_specs=[
            pl.BlockSpec((G, DH), lambda: (0, 0)),
            pl.BlockSpec((G, DH), lambda: (0, 0)),
            pl.BlockSpec((G, DH), lambda: (0, 0)),
            pl.BlockSpec((D, D), lambda: (0, 0)),
            pl.BlockSpec((1, D), lambda: (0, 0)),
            pl.BlockSpec((D, 2), lambda: (0, 0)),
            pl.BlockSpec((1, 2), lambda: (0, 0)),
        ],
        out_specs=pl.BlockSpec((G, 2), lambda: (0, 0)),
        out_shape=jax.ShapeDtypeStruct((G, 2), jnp.float32),
    )(sums_a, sums_b, counts, shared_W, shared_b, W_heads, b_heads)


@jax.jit
def kernel(x, edge_index, batch, proj_W, proj_b,
           conv0_W1, conv0_b1, conv0_W2, conv0_b2,
           conv1_W1, conv1_b1, conv1_W2, conv1_b2,
           conv2_W1, conv2_b1, conv2_W2, conv2_b2,
           shared_W, shared_b, energy_W, energy_b, dipole_W, dipole_b):
    xp = jnp.pad(x, ((0, NP - N), (0, 0)))
    srcp = jnp.concatenate(
        [edge_index[0], jnp.zeros((EPAD - E,), jnp.int32)])
    dstp = jnp.concatenate(
        [edge_index[1], jnp.full((EPAD - E,), N, jnp.int32)])
    batchp = jnp.concatenate([batch, jnp.full((NP - N,), G, jnp.int32)])
    W_heads = jnp.concatenate([energy_W, dipole_W], axis=1)
    b_heads = jnp.concatenate([energy_b, dipole_b]).reshape(1, 2)

    h_a, h_b = _tc_proj(xp, proj_W, proj_b.reshape(1, D))

    convs = [(conv0_W1, conv0_b1, conv0_W2, conv0_b2),
             (conv1_W1, conv1_b1, conv1_W2, conv1_b2),
             (conv2_W1, conv2_b1, conv2_W2, conv2_b2)]
    for (W1, b1, W2, b2) in convs:
        agg_a, agg_b = _sc_edge_agg(h_a, h_b, srcp, dstp)
        h_a, h_b = _tc_mlp(h_a, h_b, agg_a, agg_b,
                           W1, b1.reshape(1, D), W2, b2.reshape(1, D))

    sums_a, sums_b, counts = _sc_pool(h_a, h_b, batchp)
    out = _tc_head(sums_a, sums_b, counts,
                   shared_W, shared_b.reshape(1, D), W_heads, b_heads)
    return (out[:, 0], out[:, 1])


# speedup vs baseline: 8.2500x; 1.0305x over previous
import functools

import jax
import jax.numpy as jnp
from jax import lax
from jax.experimental import pallas as pl
from jax.experimental.pallas import tpu as pltpu
from jax.experimental.pallas import tpu_sc as plsc

_SC_PARAMS = pltpu.CompilerParams(use_tc_tiling_on_sc=False)

N = 50000
E = 800000
DIN = 4
D = 64
DH = 32
G = 512

NSUB = 16
NCORE = 2

NP = 51200
EPAD = 802816
CH = 128
ECHUNKS = EPAD // (NSUB * CH)
NGRP = ECHUNKS // 2
NCHUNKS = NP // (NSUB * CH)
GP = 528

BLK = 1024
NBLK = NP // BLK


def _sc_edge_agg(h_a, h_b, srcp, dstp):
    mesh = plsc.VectorSubcoreMesh(core_axis_name="c", subcore_axis_name="s")

    @functools.partial(
        pl.kernel,
        out_type=[jax.ShapeDtypeStruct((NP, DH), jnp.float32),
                  jax.ShapeDtypeStruct((NP, DH), jnp.float32)],
        mesh=mesh,
        compiler_params=_SC_PARAMS,
        scratch_types=[
            pltpu.VMEM_SHARED((NP, DH), jnp.float32),
            pltpu.VMEM((8, CH), jnp.int32),
            pltpu.VMEM((8, CH), jnp.int32),
            pltpu.VMEM((CH, DH), jnp.float32),
            pltpu.VMEM((CH, DH), jnp.float32),
            pltpu.VMEM((CH, DH), jnp.float32),
            pltpu.VMEM((CH, DH), jnp.float32),
            pltpu.VMEM((CH, DH), jnp.float32),
        ] + [pltpu.SemaphoreType.DMA] * 20,
    )
    def k(ha_hbm, hb_hbm, src_hbm, dst_hbm, oa_hbm, ob_hbm,
          acc, idxs, idxd, rows0, rows1, rows2, rows3, zbuf, *sems):
        c = lax.axis_index("c")
        s = lax.axis_index("s")
        sem_i = sems[0:8]
        sem_g = sems[8:12]
        sem_s = sems[12:20]

        @pl.loop(0, CH)
        def _(r):
            @pl.loop(0, DH // 16)
            def _(j):
                zbuf[r, pl.ds(j * 16, 16)] = jnp.zeros((16,), jnp.float32)

        @pl.loop(0, NCHUNKS)
        def _(kk):
            pltpu.sync_copy(zbuf, acc.at[pl.ds((s * NCHUNKS + kk) * CH, CH)])

        plsc.subcore_barrier()

        rows = (rows0, rows1, rows2, rows3)

        def edge_loop(h_hbm):
            base = s * (ECHUNKS * CH)

            def fire_idx(j, b):
                off = base + j * CH
                pltpu.async_copy(src_hbm.at[pl.ds(off, CH)], idxs.at[b],
                                 sem_i[b])
                pltpu.async_copy(dst_hbm.at[pl.ds(off, CH)], idxd.at[b],
                                 sem_i[b])

            def wait_idx(j, b):
                off = base + j * CH
                pltpu.make_async_copy(src_hbm.at[pl.ds(off, CH)], idxs.at[b],
                                      sem_i[b]).wait()
                pltpu.make_async_copy(dst_hbm.at[pl.ds(off, CH)], idxd.at[b],
                                      sem_i[b]).wait()

            def fire_gather(bi, br):
                pltpu.async_copy(h_hbm.at[idxs.at[bi]], rows[br], sem_g[br])

            def wait_gather(bi, br):
                pltpu.make_async_copy(h_hbm.at[idxs.at[bi]], rows[br],
                                      sem_g[br]).wait()

            fire_idx(0, 0)
            fire_idx(1, 1)
            fire_idx(2, 2)
            wait_idx(0, 0)
            fire_gather(0, 0)
            wait_idx(1, 1)
            fire_gather(1, 1)

            @pl.loop(0, ECHUNKS // 8)
            def _(it):
                handles = [None] * 8
                for m in range(8):
                    k_ = 8 * it + m
                    bi = m % 8
                    br = m % 4
                    wait_gather(bi, br)

                    if m >= 2:
                        handles[m - 2].wait()

                    @pl.when(k_ + 2 < ECHUNKS)
                    def _():
                        wait_idx(k_ + 2, (bi + 2) % 8)
                        fire_gather((bi + 2) % 8, (br + 2) % 4)

                    handles[m] = pltpu.async_copy(
                        rows[br], acc.at[idxd.at[bi]], sem_s[m], add=True)

                    @pl.when(k_ + 3 < ECHUNKS)
                    def _():
                        fire_idx(k_ + 3, (bi + 3) % 8)

                handles[6].wait()
                handles[7].wait()

        @pl.when(c == 0)
        def _():
            edge_loop(ha_hbm)

        @pl.when(c == 1)
        def _():
            edge_loop(hb_hbm)

        plsc.subcore_barrier()

        def copy_out(o_hbm):
            @pl.loop(0, NCHUNKS)
            def _(kk):
                r0 = (s * NCHUNKS + kk) * CH
                pltpu.sync_copy(acc.at[pl.ds(r0, CH)], o_hbm.at[pl.ds(r0, CH)])

        @pl.when(c == 0)
        def _():
            copy_out(oa_hbm)

        @pl.when(c == 1)
        def _():
            copy_out(ob_hbm)

    return k(h_a, h_b, srcp, dstp)


def _dot_f32x3(a, b):
    return jnp.dot(a.astype(jnp.bfloat16), b.astype(jnp.bfloat16),
                   preferred_element_type=jnp.float32)


def _tc_proj(xp, proj_W, proj_b):
    def body(x_ref, w_ref, b_ref, oa_ref, ob_ref):
        h = _dot_f32x3(x_ref[...], w_ref[...]) + b_ref[...]
        oa_ref[...] = h[:, :DH]
        ob_ref[...] = h[:, DH:]

    return pl.pallas_call(
        body,
        grid=(NBLK,),
        in_specs=[
            pl.BlockSpec((BLK, DIN), lambda i: (i, 0)),
            pl.BlockSpec((DIN, D), lambda i: (0, 0)),
            pl.BlockSpec((1, D), lambda i: (0, 0)),
        ],
        out_specs=[
            pl.BlockSpec((BLK, DH), lambda i: (i, 0)),
            pl.BlockSpec((BLK, DH), lambda i: (i, 0)),
        ],
        out_shape=[jax.ShapeDtypeStruct((NP, DH), jnp.float32),
                   jax.ShapeDtypeStruct((NP, DH), jnp.float32)],
    )(xp, proj_W, proj_b)


def _tc_mlp(h_a, h_b, agg_a, agg_b, W1, b1, W2, b2):
    def body(ha_ref, hb_ref, aa_ref, ab_ref, w1_ref, b1_ref, w2_ref, b2_ref,
             oa_ref, ob_ref):
        z = jnp.concatenate(
            [ha_ref[...] + aa_ref[...], hb_ref[...] + ab_ref[...]], axis=1)
        y = jnp.maximum(_dot_f32x3(z, w1_ref[...]) + b1_ref[...], 0.0)
        o = jnp.maximum(_dot_f32x3(y, w2_ref[...]) + b2_ref[...], 0.0)
        oa_ref[...] = o[:, :DH]
        ob_ref[...] = o[:, DH:]

    row = pl.BlockSpec((BLK, DH), lambda i: (i, 0))
    return pl.pallas_call(
        body,
        grid=(NBLK,),
        in_specs=[
            row, row, row, row,
            pl.BlockSpec((D, D), lambda i: (0, 0)),
            pl.BlockSpec((1, D), lambda i: (0, 0)),
            pl.BlockSpec((D, D), lambda i: (0, 0)),
            pl.BlockSpec((1, D), lambda i: (0, 0)),
        ],
        out_specs=[row, row],
        out_shape=[jax.ShapeDtypeStruct((NP, DH), jnp.float32),
                   jax.ShapeDtypeStruct((NP, DH), jnp.float32)],
    )(h_a, h_b, agg_a, agg_b, W1, b1, W2, b2)


def _tc_pool_head(h_a, h_b, batch3, shared_W, shared_b, W_heads, b_heads):
    def body(ha_ref, hb_ref, b_ref, w_ref, bs_ref, wh_ref, bh_ref,
             o_ref, acc_ref):
        i = pl.program_id(0)

        @pl.when(i == 0)
        def _():
            acc_ref[...] = jnp.zeros_like(acc_ref)

        bids = b_ref[0].astype(jnp.int32)
        onehot_t = jnp.where(
            jax.lax.broadcast_in_dim(bids, (G, BLK), (0, 1)) ==
            jax.lax.broadcasted_iota(jnp.int32, (G, BLK), 0),
            1.0, 0.0)
        hx = jnp.concatenate(
            [ha_ref[...], hb_ref[...], jnp.ones((BLK, 1), jnp.float32)],
            axis=1)
        acc_ref[:, :D + 1] += jnp.dot(onehot_t, hx,
                                      preferred_element_type=jnp.float32)

        @pl.when(i == NBLK - 1)
        def _():
            sums = acc_ref[:, :D]
            cnt = jnp.maximum(acc_ref[:, D:D + 1], 1.0)
            pooled = sums / cnt
            s = jnp.maximum(_dot_f32x3(pooled, w_ref[...]) + bs_ref[...], 0.0)
            o_ref[...] = _dot_f32x3(s, wh_ref[...]) + bh_ref[...]

    return pl.pallas_call(
        body,
        grid=(NBLK,),
        in_specs=[
            pl.BlockSpec((BLK, DH), lambda i: (i, 0)),
            pl.BlockSpec((BLK, DH), lambda i: (i, 0)),
            pl.BlockSpec((1, 1, BLK), lambda i: (i, 0, 0)),
            pl.BlockSpec((D, D), lambda i: (0, 0)),
            pl.BlockSpec((1, D), lambda i: (0, 0)),
            pl.BlockSpec((D, 2), lambda i: (0, 0)),
            pl.BlockSpec((1, 2), lambda i: (0, 0)),
        ],
        out_specs=pl.BlockSpec((G, 2), lambda i: (0, 0)),
        out_shape=jax.ShapeDtypeStruct((G, 2), jnp.float32),
        scratch_shapes=[pltpu.VMEM((G, 128), jnp.float32)],
    )(h_a, h_b, batch3, shared_W, shared_b, W_heads, b_heads)


@jax.jit
def kernel(x, edge_index, batch, proj_W, proj_b,
           conv0_W1, conv0_b1, conv0_W2, conv0_b2,
           conv1_W1, conv1_b1, conv1_W2, conv1_b2,
           conv2_W1, conv2_b1, conv2_W2, conv2_b2,
           shared_W, shared_b, energy_W, energy_b, dipole_W, dipole_b):
    xp = jnp.pad(x, ((0, NP - N), (0, 0)))
    srcp = jnp.concatenate(
        [edge_index[0], jnp.zeros((EPAD - E,), jnp.int32)])
    dstp = jnp.concatenate(
        [edge_index[1], jnp.full((EPAD - E,), N, jnp.int32)])
    batch3 = jnp.concatenate(
        [batch, jnp.full((NP - N,), G, jnp.int32)]
    ).astype(jnp.float32).reshape(NBLK, 1, BLK)
    W_heads = jnp.concatenate([energy_W, dipole_W], axis=1)
    b_heads = jnp.concatenate([energy_b, dipole_b]).reshape(1, 2)

    h_a, h_b = _tc_proj(xp, proj_W, proj_b.reshape(1, D))

    convs = [(conv0_W1, conv0_b1, conv0_W2, conv0_b2),
             (conv1_W1, conv1_b1, conv1_W2, conv1_b2),
             (conv2_W1, conv2_b1, conv2_W2, conv2_b2)]
    for (W1, b1, W2, b2) in convs:
        agg_a, agg_b = _sc_edge_agg(h_a, h_b, srcp, dstp)
        h_a, h_b = _tc_mlp(h_a, h_b, agg_a, agg_b,
                           W1, b1.reshape(1, D), W2, b2.reshape(1, D))

    out = _tc_pool_head(h_a, h_b, batch3,
                        shared_W, shared_b.reshape(1, D), W_heads, b_heads)
    return (out[:, 0], out[:, 1])


# speedup vs baseline: 8.7766x; 1.0638x over previous
import functools

import jax
import jax.numpy as jnp
from jax import lax
from jax.experimental import pallas as pl
from jax.experimental.pallas import tpu as pltpu
from jax.experimental.pallas import tpu_sc as plsc

_SC_PARAMS = pltpu.CompilerParams(use_tc_tiling_on_sc=False)

N = 50000
E = 800000
DIN = 4
D = 64
DH = 32
G = 512

NSUB = 16
NCORE = 2

NP = 51200
EPAD = 802816
CH = 128
ECHUNKS = EPAD // (NSUB * CH)
NGRP = ECHUNKS // 2
NCHUNKS = NP // (NSUB * CH)
GP = 528

BLK = 1024
NBLK = NP // BLK
BLKM = 6400
NBLKM = NP // BLKM


def _sc_edge_agg(h_a, h_b, srcp, dstp):
    mesh = plsc.VectorSubcoreMesh(core_axis_name="c", subcore_axis_name="s")

    @functools.partial(
        pl.kernel,
        out_type=[jax.ShapeDtypeStruct((NP, DH), jnp.float32),
                  jax.ShapeDtypeStruct((NP, DH), jnp.float32)],
        mesh=mesh,
        compiler_params=_SC_PARAMS,
        scratch_types=[
            pltpu.VMEM_SHARED((NP, DH), jnp.float32),
            pltpu.VMEM((8, CH), jnp.int32),
            pltpu.VMEM((8, CH), jnp.int32),
            pltpu.VMEM((CH, DH), jnp.float32),
            pltpu.VMEM((CH, DH), jnp.float32),
            pltpu.VMEM((CH, DH), jnp.float32),
            pltpu.VMEM((CH, DH), jnp.float32),
            pltpu.VMEM((CH, DH), jnp.float32),
        ] + [pltpu.SemaphoreType.DMA] * 20,
    )
    def k(ha_hbm, hb_hbm, src_hbm, dst_hbm, oa_hbm, ob_hbm,
          acc, idxs, idxd, rows0, rows1, rows2, rows3, zbuf, *sems):
        c = lax.axis_index("c")
        s = lax.axis_index("s")
        sem_i = sems[0:8]
        sem_g = sems[8:12]
        sem_s = sems[12:20]

        @pl.loop(0, CH)
        def _(r):
            @pl.loop(0, DH // 16)
            def _(j):
                zbuf[r, pl.ds(j * 16, 16)] = jnp.zeros((16,), jnp.float32)

        @pl.loop(0, NCHUNKS)
        def _(kk):
            pltpu.sync_copy(zbuf, acc.at[pl.ds((s * NCHUNKS + kk) * CH, CH)])

        plsc.subcore_barrier()

        rows = (rows0, rows1, rows2, rows3)

        def edge_loop(h_hbm):
            base = s * (ECHUNKS * CH)

            def fire_idx(j, b):
                off = base + j * CH
                pltpu.async_copy(src_hbm.at[pl.ds(off, CH)], idxs.at[b],
                                 sem_i[b])
                pltpu.async_copy(dst_hbm.at[pl.ds(off, CH)], idxd.at[b],
                                 sem_i[b])

            def wait_idx(j, b):
                off = base + j * CH
                pltpu.make_async_copy(src_hbm.at[pl.ds(off, CH)], idxs.at[b],
                                      sem_i[b]).wait()
                pltpu.make_async_copy(dst_hbm.at[pl.ds(off, CH)], idxd.at[b],
                                      sem_i[b]).wait()

            def fire_gather(bi, br):
                pltpu.async_copy(h_hbm.at[idxs.at[bi]], rows[br], sem_g[br])

            def wait_gather(bi, br):
                pltpu.make_async_copy(h_hbm.at[idxs.at[bi]], rows[br],
                                      sem_g[br]).wait()

            fire_idx(0, 0)
            fire_idx(1, 1)
            fire_idx(2, 2)
            wait_idx(0, 0)
            fire_gather(0, 0)
            wait_idx(1, 1)
            fire_gather(1, 1)

            @pl.loop(0, ECHUNKS // 8)
            def _(it):
                handles = [None] * 8
                for m in range(8):
                    k_ = 8 * it + m
                    bi = m % 8
                    br = m % 4
                    wait_gather(bi, br)

                    if m >= 2:
                        handles[m - 2].wait()

                    @pl.when(k_ + 2 < ECHUNKS)
                    def _():
                        wait_idx(k_ + 2, (bi + 2) % 8)
                        fire_gather((bi + 2) % 8, (br + 2) % 4)

                    handles[m] = pltpu.async_copy(
                        rows[br], acc.at[idxd.at[bi]], sem_s[m], add=True)

                    @pl.when(k_ + 3 < ECHUNKS)
                    def _():
                        fire_idx(k_ + 3, (bi + 3) % 8)

                handles[6].wait()
                handles[7].wait()

        @pl.when(c == 0)
        def _():
            edge_loop(ha_hbm)

        @pl.when(c == 1)
        def _():
            edge_loop(hb_hbm)

        plsc.subcore_barrier()

        def copy_out(o_hbm):
            @pl.loop(0, NCHUNKS)
            def _(kk):
                r0 = (s * NCHUNKS + kk) * CH
                pltpu.sync_copy(acc.at[pl.ds(r0, CH)], o_hbm.at[pl.ds(r0, CH)])

        @pl.when(c == 0)
        def _():
            copy_out(oa_hbm)

        @pl.when(c == 1)
        def _():
            copy_out(ob_hbm)

    return k(h_a, h_b, srcp, dstp)


def _dot_f32x3(a, b):
    return jnp.dot(a.astype(jnp.bfloat16), b.astype(jnp.bfloat16),
                   preferred_element_type=jnp.float32)


def _tc_proj(xp, proj_W, proj_b):
    def body(x_ref, w_ref, b_ref, oa_ref, ob_ref):
        h = _dot_f32x3(x_ref[...], w_ref[...]) + b_ref[...]
        oa_ref[...] = h[:, :DH]
        ob_ref[...] = h[:, DH:]

    return pl.pallas_call(
        body,
        grid=(NBLKM,),
        in_specs=[
            pl.BlockSpec((BLKM, DIN), lambda i: (i, 0)),
            pl.BlockSpec((DIN, D), lambda i: (0, 0)),
            pl.BlockSpec((1, D), lambda i: (0, 0)),
        ],
        out_specs=[
            pl.BlockSpec((BLKM, DH), lambda i: (i, 0)),
            pl.BlockSpec((BLKM, DH), lambda i: (i, 0)),
        ],
        out_shape=[jax.ShapeDtypeStruct((NP, DH), jnp.float32),
                   jax.ShapeDtypeStruct((NP, DH), jnp.float32)],
    )(xp, proj_W, proj_b)


def _tc_mlp(h_a, h_b, agg_a, agg_b, W1, b1, W2, b2):
    def body(ha_ref, hb_ref, aa_ref, ab_ref, w1_ref, b1_ref, w2_ref, b2_ref,
             oa_ref, ob_ref):
        z = jnp.concatenate(
            [ha_ref[...] + aa_ref[...], hb_ref[...] + ab_ref[...]], axis=1)
        y = jnp.maximum(_dot_f32x3(z, w1_ref[...]) + b1_ref[...], 0.0)
        o = jnp.maximum(_dot_f32x3(y, w2_ref[...]) + b2_ref[...], 0.0)
        oa_ref[...] = o[:, :DH]
        ob_ref[...] = o[:, DH:]

    row = pl.BlockSpec((BLKM, DH), lambda i: (i, 0))
    return pl.pallas_call(
        body,
        grid=(NBLKM,),
        in_specs=[
            row, row, row, row,
            pl.BlockSpec((D, D), lambda i: (0, 0)),
            pl.BlockSpec((1, D), lambda i: (0, 0)),
            pl.BlockSpec((D, D), lambda i: (0, 0)),
            pl.BlockSpec((1, D), lambda i: (0, 0)),
        ],
        out_specs=[row, row],
        out_shape=[jax.ShapeDtypeStruct((NP, DH), jnp.float32),
                   jax.ShapeDtypeStruct((NP, DH), jnp.float32)],
    )(h_a, h_b, agg_a, agg_b, W1, b1, W2, b2)


def _tc_pool_head(h_a, h_b, batch3, shared_W, shared_b, W_heads, b_heads):
    def body(ha_ref, hb_ref, b_ref, w_ref, bs_ref, wh_ref, bh_ref,
             o_ref, acc_ref):
        i = pl.program_id(0)

        @pl.when(i == 0)
        def _():
            acc_ref[...] = jnp.zeros_like(acc_ref)

        bids = b_ref[0].astype(jnp.int32)
        onehot_t = jnp.where(
            jax.lax.broadcast_in_dim(bids, (G, BLK), (0, 1)) ==
            jax.lax.broadcasted_iota(jnp.int32, (G, BLK), 0),
            1.0, 0.0)
        hx = jnp.concatenate(
            [ha_ref[...], hb_ref[...], jnp.ones((BLK, 1), jnp.float32)],
            axis=1)
        acc_ref[:, :D + 1] += jnp.dot(onehot_t, hx,
                                      preferred_element_type=jnp.float32)

        @pl.when(i == NBLK - 1)
        def _():
            sums = acc_ref[:, :D]
            cnt = jnp.maximum(acc_ref[:, D:D + 1], 1.0)
            pooled = sums / cnt
            s = jnp.maximum(_dot_f32x3(pooled, w_ref[...]) + bs_ref[...], 0.0)
            o_ref[...] = _dot_f32x3(s, wh_ref[...]) + bh_ref[...]

    return pl.pallas_call(
        body,
        grid=(NBLK,),
        in_specs=[
            pl.BlockSpec((BLK, DH), lambda i: (i, 0)),
            pl.BlockSpec((BLK, DH), lambda i: (i, 0)),
            pl.BlockSpec((1, 1, BLK), lambda i: (i, 0, 0)),
            pl.BlockSpec((D, D), lambda i: (0, 0)),
            pl.BlockSpec((1, D), lambda i: (0, 0)),
            pl.BlockSpec((D, 2), lambda i: (0, 0)),
            pl.BlockSpec((1, 2), lambda i: (0, 0)),
        ],
        out_specs=pl.BlockSpec((G, 2), lambda i: (0, 0)),
        out_shape=jax.ShapeDtypeStruct((G, 2), jnp.float32),
        scratch_shapes=[pltpu.VMEM((G, 128), jnp.float32)],
    )(h_a, h_b, batch3, shared_W, shared_b, W_heads, b_heads)


@jax.jit
def kernel(x, edge_index, batch, proj_W, proj_b,
           conv0_W1, conv0_b1, conv0_W2, conv0_b2,
           conv1_W1, conv1_b1, conv1_W2, conv1_b2,
           conv2_W1, conv2_b1, conv2_W2, conv2_b2,
           shared_W, shared_b, energy_W, energy_b, dipole_W, dipole_b):
    xp = jnp.pad(x, ((0, NP - N), (0, 0)))
    srcp = jnp.concatenate(
        [edge_index[0], jnp.zeros((EPAD - E,), jnp.int32)])
    dstp = jnp.concatenate(
        [edge_index[1], jnp.full((EPAD - E,), N, jnp.int32)])
    batch3 = jnp.concatenate(
        [batch, jnp.full((NP - N,), G, jnp.int32)]
    ).astype(jnp.float32).reshape(NBLK, 1, BLK)
    W_heads = jnp.concatenate([energy_W, dipole_W], axis=1)
    b_heads = jnp.concatenate([energy_b, dipole_b]).reshape(1, 2)

    h_a, h_b = _tc_proj(xp, proj_W, proj_b.reshape(1, D))

    convs = [(conv0_W1, conv0_b1, conv0_W2, conv0_b2),
             (conv1_W1, conv1_b1, conv1_W2, conv1_b2),
             (conv2_W1, conv2_b1, conv2_W2, conv2_b2)]
    for (W1, b1, W2, b2) in convs:
        agg_a, agg_b = _sc_edge_agg(h_a, h_b, srcp, dstp)
        h_a, h_b = _tc_mlp(h_a, h_b, agg_a, agg_b,
                           W1, b1.reshape(1, D), W2, b2.reshape(1, D))

    out = _tc_pool_head(h_a, h_b, batch3,
                        shared_W, shared_b.reshape(1, D), W_heads, b_heads)
    return (out[:, 0], out[:, 1])


# speedup vs baseline: 8.9341x; 1.0179x over previous
import functools

import jax
import jax.numpy as jnp
from jax import lax
from jax.experimental import pallas as pl
from jax.experimental.pallas import tpu as pltpu
from jax.experimental.pallas import tpu_sc as plsc

_SC_PARAMS = pltpu.CompilerParams(use_tc_tiling_on_sc=False)

N = 50000
E = 800000
DIN = 4
D = 64
DH = 32
G = 512

NSUB = 16
NCORE = 2

NP = 51200
EPAD = 802816
CH = 128
ECHUNKS = EPAD // (NSUB * CH)
NGRP = ECHUNKS // 2
NCHUNKS = NP // (NSUB * CH)
GP = 528

BLK = 1024
NBLK = NP // BLK
BLKM = 6400
NBLKM = NP // BLKM


def _sc_edge_agg(h_a, h_b, srcp, dstp):
    mesh = plsc.VectorSubcoreMesh(core_axis_name="c", subcore_axis_name="s")

    @functools.partial(
        pl.kernel,
        out_type=[jax.ShapeDtypeStruct((NP, DH), jnp.float32),
                  jax.ShapeDtypeStruct((NP, DH), jnp.float32)],
        mesh=mesh,
        compiler_params=_SC_PARAMS,
        scratch_types=[
            pltpu.VMEM_SHARED((NP, DH), jnp.float32),
            pltpu.VMEM((8, CH), jnp.int32),
            pltpu.VMEM((8, CH), jnp.int32),
            pltpu.VMEM((CH, DH), jnp.float32),
            pltpu.VMEM((CH, DH), jnp.float32),
            pltpu.VMEM((CH, DH), jnp.float32),
            pltpu.VMEM((CH, DH), jnp.float32),
            pltpu.VMEM((CH, DH), jnp.float32),
        ] + [pltpu.SemaphoreType.DMA] * 20,
    )
    def k(ha_hbm, hb_hbm, src_hbm, dst_hbm, oa_hbm, ob_hbm,
          acc, idxs, idxd, rows0, rows1, rows2, rows3, zbuf, *sems):
        c = lax.axis_index("c")
        s = lax.axis_index("s")
        sem_i = sems[0:8]
        sem_g = sems[8:12]
        sem_s = sems[12:20]

        @pl.loop(0, CH)
        def _(r):
            @pl.loop(0, DH // 16)
            def _(j):
                zbuf[r, pl.ds(j * 16, 16)] = jnp.zeros((16,), jnp.float32)

        zsems = sems[0:4]
        for kk in range(NCHUNKS):
            pltpu.async_copy(zbuf, acc.at[pl.ds((s * NCHUNKS + kk) * CH, CH)],
                             zsems[kk % 4])
            if kk >= 4:
                pltpu.make_async_copy(
                    zbuf, acc.at[pl.ds((s * NCHUNKS + kk - 4) * CH, CH)],
                    zsems[kk % 4]).wait()
        for kk in range(NCHUNKS - 4, NCHUNKS):
            pltpu.make_async_copy(
                zbuf, acc.at[pl.ds((s * NCHUNKS + kk) * CH, CH)],
                zsems[kk % 4]).wait()

        plsc.subcore_barrier()

        rows = (rows0, rows1, rows2, rows3)

        def edge_loop(h_hbm):
            base = s * (ECHUNKS * CH)

            def fire_idx(j, b):
                off = base + j * CH
                pltpu.async_copy(src_hbm.at[pl.ds(off, CH)], idxs.at[b],
                                 sem_i[b])
                pltpu.async_copy(dst_hbm.at[pl.ds(off, CH)], idxd.at[b],
                                 sem_i[b])

            def wait_idx(j, b):
                off = base + j * CH
                pltpu.make_async_copy(src_hbm.at[pl.ds(off, CH)], idxs.at[b],
                                      sem_i[b]).wait()
                pltpu.make_async_copy(dst_hbm.at[pl.ds(off, CH)], idxd.at[b],
                                      sem_i[b]).wait()

            def fire_gather(bi, br):
                pltpu.async_copy(h_hbm.at[idxs.at[bi]], rows[br], sem_g[br])

            def wait_gather(bi, br):
                pltpu.make_async_copy(h_hbm.at[idxs.at[bi]], rows[br],
                                      sem_g[br]).wait()

            fire_idx(0, 0)
            fire_idx(1, 1)
            fire_idx(2, 2)
            wait_idx(0, 0)
            fire_gather(0, 0)
            wait_idx(1, 1)
            fire_gather(1, 1)

            @pl.loop(0, ECHUNKS // 8)
            def _(it):
                handles = [None] * 8
                for m in range(8):
                    k_ = 8 * it + m
                    bi = m % 8
                    br = m % 4
                    wait_gather(bi, br)

                    if m >= 2:
                        handles[m - 2].wait()

                    @pl.when(k_ + 2 < ECHUNKS)
                    def _():
                        wait_idx(k_ + 2, (bi + 2) % 8)
                        fire_gather((bi + 2) % 8, (br + 2) % 4)

                    handles[m] = pltpu.async_copy(
                        rows[br], acc.at[idxd.at[bi]], sem_s[m], add=True)

                    @pl.when(k_ + 3 < ECHUNKS)
                    def _():
                        fire_idx(k_ + 3, (bi + 3) % 8)

                handles[6].wait()
                handles[7].wait()

        @pl.when(c == 0)
        def _():
            edge_loop(ha_hbm)

        @pl.when(c == 1)
        def _():
            edge_loop(hb_hbm)

        plsc.subcore_barrier()

        def copy_out(o_hbm):
            hs = []
            for q in range(4):
                r0 = s * (NCHUNKS * CH) + q * 800
                hs.append(pltpu.async_copy(acc.at[pl.ds(r0, 800)],
                                           o_hbm.at[pl.ds(r0, 800)],
                                           sems[q]))
            for h in hs:
                h.wait()

        @pl.when(c == 0)
        def _():
            copy_out(oa_hbm)

        @pl.when(c == 1)
        def _():
            copy_out(ob_hbm)

    return k(h_a, h_b, srcp, dstp)


def _dot_f32x3(a, b):
    return jnp.dot(a.astype(jnp.bfloat16), b.astype(jnp.bfloat16),
                   preferred_element_type=jnp.float32)


def _tc_proj(xp, proj_W, proj_b):
    def body(x_ref, w_ref, b_ref, oa_ref, ob_ref):
        h = _dot_f32x3(x_ref[...], w_ref[...]) + b_ref[...]
        oa_ref[...] = h[:, :DH]
        ob_ref[...] = h[:, DH:]

    return pl.pallas_call(
        body,
        grid=(NBLKM,),
        in_specs=[
            pl.BlockSpec((BLKM, DIN), lambda i: (i, 0)),
            pl.BlockSpec((DIN, D), lambda i: (0, 0)),
            pl.BlockSpec((1, D), lambda i: (0, 0)),
        ],
        out_specs=[
            pl.BlockSpec((BLKM, DH), lambda i: (i, 0)),
            pl.BlockSpec((BLKM, DH), lambda i: (i, 0)),
        ],
        out_shape=[jax.ShapeDtypeStruct((NP, DH), jnp.float32),
                   jax.ShapeDtypeStruct((NP, DH), jnp.float32)],
    )(xp, proj_W, proj_b)


def _tc_mlp(h_a, h_b, agg_a, agg_b, W1, b1, W2, b2):
    def body(ha_ref, hb_ref, aa_ref, ab_ref, w1_ref, b1_ref, w2_ref, b2_ref,
             oa_ref, ob_ref):
        z = jnp.concatenate(
            [ha_ref[...] + aa_ref[...], hb_ref[...] + ab_ref[...]], axis=1)
        y = jnp.maximum(_dot_f32x3(z, w1_ref[...]) + b1_ref[...], 0.0)
        o = jnp.maximum(_dot_f32x3(y, w2_ref[...]) + b2_ref[...], 0.0)
        oa_ref[...] = o[:, :DH]
        ob_ref[...] = o[:, DH:]

    row = pl.BlockSpec((BLKM, DH), lambda i: (i, 0))
    return pl.pallas_call(
        body,
        grid=(NBLKM,),
        in_specs=[
            row, row, row, row,
            pl.BlockSpec((D, D), lambda i: (0, 0)),
            pl.BlockSpec((1, D), lambda i: (0, 0)),
            pl.BlockSpec((D, D), lambda i: (0, 0)),
            pl.BlockSpec((1, D), lambda i: (0, 0)),
        ],
        out_specs=[row, row],
        out_shape=[jax.ShapeDtypeStruct((NP, DH), jnp.float32),
                   jax.ShapeDtypeStruct((NP, DH), jnp.float32)],
    )(h_a, h_b, agg_a, agg_b, W1, b1, W2, b2)


def _tc_pool_head(h_a, h_b, batch3, shared_W, shared_b, W_heads, b_heads):
    def body(ha_ref, hb_ref, b_ref, w_ref, bs_ref, wh_ref, bh_ref,
             o_ref, acc_ref):
        i = pl.program_id(0)

        @pl.when(i == 0)
        def _():
            acc_ref[...] = jnp.zeros_like(acc_ref)

        bids = b_ref[0].astype(jnp.int32)
        onehot_t = jnp.where(
            jax.lax.broadcast_in_dim(bids, (G, BLK), (0, 1)) ==
            jax.lax.broadcasted_iota(jnp.int32, (G, BLK), 0),
            1.0, 0.0)
        hx = jnp.concatenate(
            [ha_ref[...], hb_ref[...], jnp.ones((BLK, 1), jnp.float32)],
            axis=1)
        acc_ref[:, :D + 1] += jnp.dot(onehot_t, hx,
                                      preferred_element_type=jnp.float32)

        @pl.when(i == NBLK - 1)
        def _():
            sums = acc_ref[:, :D]
            cnt = jnp.maximum(acc_ref[:, D:D + 1], 1.0)
            pooled = sums / cnt
            s = jnp.maximum(_dot_f32x3(pooled, w_ref[...]) + bs_ref[...], 0.0)
            o_ref[...] = _dot_f32x3(s, wh_ref[...]) + bh_ref[...]

    return pl.pallas_call(
        body,
        grid=(NBLK,),
        in_specs=[
            pl.BlockSpec((BLK, DH), lambda i: (i, 0)),
            pl.BlockSpec((BLK, DH), lambda i: (i, 0)),
            pl.BlockSpec((1, 1, BLK), lambda i: (i, 0, 0)),
            pl.BlockSpec((D, D), lambda i: (0, 0)),
            pl.BlockSpec((1, D), lambda i: (0, 0)),
            pl.BlockSpec((D, 2), lambda i: (0, 0)),
            pl.BlockSpec((1, 2), lambda i: (0, 0)),
        ],
        out_specs=pl.BlockSpec((G, 2), lambda i: (0, 0)),
        out_shape=jax.ShapeDtypeStruct((G, 2), jnp.float32),
        scratch_shapes=[pltpu.VMEM((G, 128), jnp.float32)],
    )(h_a, h_b, batch3, shared_W, shared_b, W_heads, b_heads)


@jax.jit
def kernel(x, edge_index, batch, proj_W, proj_b,
           conv0_W1, conv0_b1, conv0_W2, conv0_b2,
           conv1_W1, conv1_b1, conv1_W2, conv1_b2,
           conv2_W1, conv2_b1, conv2_W2, conv2_b2,
           shared_W, shared_b, energy_W, energy_b, dipole_W, dipole_b):
    xp = jnp.pad(x, ((0, NP - N), (0, 0)))
    srcp = jnp.concatenate(
        [edge_index[0], jnp.zeros((EPAD - E,), jnp.int32)])
    dstp = jnp.concatenate(
        [edge_index[1], jnp.full((EPAD - E,), N, jnp.int32)])
    batch3 = jnp.concatenate(
        [batch, jnp.full((NP - N,), G, jnp.int32)]
    ).astype(jnp.float32).reshape(NBLK, 1, BLK)
    W_heads = jnp.concatenate([energy_W, dipole_W], axis=1)
    b_heads = jnp.concatenate([energy_b, dipole_b]).reshape(1, 2)

    h_a, h_b = _tc_proj(xp, proj_W, proj_b.reshape(1, D))

    convs = [(conv0_W1, conv0_b1, conv0_W2, conv0_b2),
             (conv1_W1, conv1_b1, conv1_W2, conv1_b2),
             (conv2_W1, conv2_b1, conv2_W2, conv2_b2)]
    for (W1, b1, W2, b2) in convs:
        agg_a, agg_b = _sc_edge_agg(h_a, h_b, srcp, dstp)
        h_a, h_b = _tc_mlp(h_a, h_b, agg_a, agg_b,
                           W1, b1.reshape(1, D), W2, b2.reshape(1, D))

    out = _tc_pool_head(h_a, h_b, batch3,
                        shared_W, shared_b.reshape(1, D), W_heads, b_heads)
    return (out[:, 0], out[:, 1])
